# baseline (speedup 1.0000x reference)
import functools

import jax
import jax.numpy as jnp
from jax import lax
from jax.experimental import pallas as pl
from jax.experimental.pallas import tpu as pltpu
from jax.experimental.pallas import tpu_sc as plsc

B, C, H, W = 16, 96, 32, 32
DIM, K = 32, 8192
N = B * H * W
R = 256
NH = N // 2
GH = NH // R


def _round_bf16(v):
    u = jax.lax.bitcast_convert_type(v, jnp.uint32)
    r = (u + jnp.uint32(0x7FFF) + ((u >> 16) & jnp.uint32(1))) & jnp.uint32(0xFFFF0000)
    return jax.lax.bitcast_convert_type(r, jnp.float32)


def _vq_block(xt_ref, wt_ref, b_ref, e_ref, ind_ref, dp_ref):
    f = jnp.dot(xt_ref[...], wt_ref[...], preferred_element_type=jnp.float32)
    f = f + b_ref[...]
    e = e_ref[...]
    f2 = jnp.sum(f * f, axis=1, keepdims=True)
    e2 = jnp.sum(e * e, axis=0, keepdims=True)
    mm2 = jnp.dot(_round_bf16(f) * -2.0, e, preferred_element_type=jnp.float32)
    d = f2 + mm2 + e2
    h = K // 2
    d0 = d[:, :h]
    d1 = d[:, h:]
    m0 = jnp.min(d0, axis=1)
    m1 = jnp.min(d1, axis=1)
    take = m1 < _round_bf16(m0)
    m_win = jnp.where(take, m1, m0)
    d_win = jnp.where(take[:, None], d1, d0)
    iota = jax.lax.broadcasted_iota(jnp.int32, (R, h), 1)
    i_win = jnp.min(jnp.where(d_win == m_win[:, None], iota, K), axis=1)
    ind_ref[0, 0, :] = i_win + jnp.where(take, h, 0)
    dp_ref[...] = jnp.sum(m_win).reshape(1, 1, 1)


def _tc_half(xt_half, wt, b2, embed):
    return pl.pallas_call(
        _vq_block,
        grid=(GH,),
        in_specs=[
            pl.BlockSpec((R, C), lambda i: (i, 0)),
            pl.BlockSpec((C, DIM), lambda i: (0, 0)),
            pl.BlockSpec((1, DIM), lambda i: (0, 0)),
            pl.BlockSpec((DIM, K), lambda i: (0, 0)),
        ],
        out_specs=[
            pl.BlockSpec((1, 1, R), lambda i: (i, 0, 0)),
            pl.BlockSpec((1, 1, 1), lambda i: (i, 0, 0)),
        ],
        out_shape=[
            jax.ShapeDtypeStruct((GH, 1, R), jnp.int32),
            jax.ShapeDtypeStruct((GH, 1, 1), jnp.float32),
        ],
    )(xt_half, wt, b2, embed)


_SC_INFO = plsc.get_sparse_core_info()
_NW = _SC_INFO.num_cores * _SC_INFO.num_subcores
_BPW = NH // _NW
_PADW = 128


@functools.partial(
    pl.kernel,
    mesh=plsc.VectorSubcoreMesh(core_axis_name="c", subcore_axis_name="s"),
    out_type=jax.ShapeDtypeStruct((NH, _PADW), jnp.float32),
    scratch_types=[
        pltpu.VMEM((_BPW,), jnp.int32),
        pltpu.VMEM((_BPW, _PADW), jnp.float32),
        pltpu.SemaphoreType.DMA,
    ],
)
def _sc_gather(table_hbm, idx_hbm, out_hbm, idx_v, rows_v, sem):
    wid = lax.axis_index("s") * _SC_INFO.num_cores + lax.axis_index("c")
    base = wid * _BPW
    pltpu.sync_copy(idx_hbm.at[pl.ds(base, _BPW)], idx_v)
    pltpu.async_copy(table_hbm.at[idx_v], rows_v, sem).wait()
    pltpu.sync_copy(rows_v, out_hbm.at[pl.ds(base, _BPW)])


def kernel(x, conv_w, conv_b, embed):
    xt = x.transpose(0, 2, 3, 1).reshape(N, C)
    wt = conv_w.T
    b2 = conv_b.reshape(1, DIM)
    table = jnp.zeros((K, _PADW), jnp.float32).at[:, :DIM].set(embed.T)
    ind3_a, dp_a = _tc_half(xt[:NH], wt, b2, embed)
    ind_a = ind3_a.reshape(NH)
    q_a = _sc_gather(table, ind_a)
    ind3_b, dp_b = _tc_half(xt[NH:], wt, b2, embed)
    ind_b = ind3_b.reshape(NH)
    q_b = _sc_gather(table, ind_b)
    q = jnp.concatenate([q_a[:, :DIM], q_b[:, :DIM]], axis=0)
    quantize = q.reshape(B, H, W, DIM).transpose(0, 3, 1, 2)
    diff = (dp_a.sum() + dp_b.sum()) / jnp.float32(N * DIM)
    embed_ind = jnp.concatenate([ind_a, ind_b]).reshape(B, H, W)
    return (quantize, diff, embed_ind)


# speedup vs baseline: 1.0560x; 1.0560x over previous
import functools

import jax
import jax.numpy as jnp
from jax import lax
from jax.experimental import pallas as pl
from jax.experimental.pallas import tpu as pltpu
from jax.experimental.pallas import tpu_sc as plsc

B, C, H, W = 16, 96, 32, 32
DIM, K = 32, 8192
N = B * H * W
R = 256
G = N // R


def _round_bf16(v):
    u = jax.lax.bitcast_convert_type(v, jnp.uint32)
    r = (u + jnp.uint32(0x7FFF) + ((u >> 16) & jnp.uint32(1))) & jnp.uint32(0xFFFF0000)
    return jax.lax.bitcast_convert_type(r, jnp.float32)


def _vq_block(xt_ref, wt_ref, b_ref, e_ref, ind_ref, dp_ref):
    f = jnp.dot(xt_ref[...], wt_ref[...], preferred_element_type=jnp.float32)
    f = f + b_ref[...]
    e = e_ref[...]
    f2 = jnp.sum(f * f, axis=1, keepdims=True)
    e2 = jnp.sum(e * e, axis=0, keepdims=True)
    mm2 = jnp.dot(_round_bf16(f) * -2.0, e, preferred_element_type=jnp.float32)
    d = f2 + mm2 + e2
    h = K // 2
    d0 = d[:, :h]
    d1 = d[:, h:]
    m0 = jnp.min(d0, axis=1)
    m1 = jnp.min(d1, axis=1)
    take = m1 < _round_bf16(m0)
    m_win = jnp.where(take, m1, m0)
    d_win = jnp.where(take[:, None], d1, d0)
    iota = jax.lax.broadcasted_iota(jnp.int32, (R, h), 1)
    i_win = jnp.min(jnp.where(d_win == m_win[:, None], iota, K), axis=1)
    ind_ref[0, 0, :] = i_win + jnp.where(take, h, 0)
    dp_ref[...] = jnp.sum(m_win).reshape(1, 1, 1)


_SC_INFO = plsc.get_sparse_core_info()
_NW = _SC_INFO.num_cores * _SC_INFO.num_subcores
_BPW = N // _NW
_PADW = 128


@functools.partial(
    pl.kernel,
    mesh=plsc.VectorSubcoreMesh(core_axis_name="c", subcore_axis_name="s"),
    out_type=jax.ShapeDtypeStruct((N, _PADW), jnp.float32),
    scratch_types=[
        pltpu.VMEM((_BPW,), jnp.int32),
        pltpu.VMEM((_BPW, _PADW), jnp.float32),
        pltpu.SemaphoreType.DMA,
    ],
)
def _sc_gather(table_hbm, idx_hbm, out_hbm, idx_v, rows_v, sem):
    wid = lax.axis_index("s") * _SC_INFO.num_cores + lax.axis_index("c")
    base = wid * _BPW
    pltpu.sync_copy(idx_hbm.at[pl.ds(base, _BPW)], idx_v)
    pltpu.async_copy(table_hbm.at[idx_v], rows_v, sem).wait()
    pltpu.sync_copy(rows_v, out_hbm.at[pl.ds(base, _BPW)])


def kernel(x, conv_w, conv_b, embed):
    xt = x.transpose(0, 2, 3, 1).reshape(N, C)
    wt = conv_w.T
    b2 = conv_b.reshape(1, DIM)
    ind3, dp = pl.pallas_call(
        _vq_block,
        grid=(G,),
        in_specs=[
            pl.BlockSpec((R, C), lambda i: (i, 0)),
            pl.BlockSpec((C, DIM), lambda i: (0, 0)),
            pl.BlockSpec((1, DIM), lambda i: (0, 0)),
            pl.BlockSpec((DIM, K), lambda i: (0, 0)),
        ],
        out_specs=[
            pl.BlockSpec((1, 1, R), lambda i: (i, 0, 0)),
            pl.BlockSpec((1, 1, 1), lambda i: (i, 0, 0)),
        ],
        out_shape=[
            jax.ShapeDtypeStruct((G, 1, R), jnp.int32),
            jax.ShapeDtypeStruct((G, 1, 1), jnp.float32),
        ],
    )(xt, wt, b2, embed)
    ind = ind3.reshape(N)
    table = jnp.zeros((K, _PADW), jnp.float32).at[:, :DIM].set(embed.T)
    q = _sc_gather(table, ind)[:, :DIM]
    quantize = q.reshape(B, H, W, DIM).transpose(0, 3, 1, 2)
    diff = dp.sum() / jnp.float32(N * DIM)
    embed_ind = ind.reshape(B, H, W)
    return (quantize, diff, embed_ind)
